# Initial kernel scaffold; baseline (speedup 1.0000x reference)
import jax
import jax.numpy as jnp
from jax.experimental import pallas as pl


def kernel(logits, e_score_correction_bias):
    raise NotImplementedError("write your pallas kernel here")


# speedup vs baseline: 2.2749x; 2.2749x over previous
import functools

import jax
import jax.numpy as jnp
from jax import lax
from jax.experimental import pallas as pl
from jax.experimental.pallas import tpu as pltpu
from jax.experimental.pallas import tpu_sc as plsc

SEQ = 16384
NE = 64
NG = 8
GSZ = NE // NG
TOPK = 8
TOPKG = 4
SCALE = 2.5
LANES = 16

_INFO = plsc.get_sparse_core_info()
NC = _INFO.num_cores
NS = _INFO.num_subcores
NW = NC * NS
TPW = SEQ // NW
NT = TPW // LANES


def _top2_combine(x1, x2, y1, y2):
    return jnp.maximum(x1, y1), jnp.maximum(jnp.minimum(x1, y1),
                                            jnp.maximum(x2, y2))


def _tree(vs, op):
    vs = list(vs)
    while len(vs) > 1:
        nxt = [op(vs[i], vs[i + 1]) for i in range(0, len(vs) - 1, 2)]
        if len(vs) % 2:
            nxt.append(vs[-1])
        vs = nxt
    return vs[0]


def _max_tree(vs):
    return _tree(vs, jnp.maximum)


def _min_tree(vs):
    return _tree(vs, jnp.minimum)


def _sum_tree(vs):
    return _tree(vs, lambda a, b: a + b)


def _router_body(logits_hbm, bias_hbm, rw_hbm, tw_hbm, ti_hbm, hist_hbm,
                 chunk, scores_t, scm_t, bias_vm, oidx, ow, counts):
    c = lax.axis_index("c")
    s = lax.axis_index("s")
    wid = s * NC + c
    base = wid * TPW

    pltpu.sync_copy(logits_hbm.at[pl.ds(base * NE, TPW * NE)], chunk)
    pltpu.sync_copy(bias_hbm, bias_vm)

    lane = lax.iota(jnp.int32, LANES)
    zeros = jnp.zeros((LANES,), jnp.float32)
    ones = jnp.ones((LANES,), jnp.float32)
    neg_inf = jnp.full((LANES,), -jnp.inf, jnp.float32)
    for i in range(NE // LANES):
        counts[pl.ds(i * LANES, LANES)] = zeros

    def tile_body(t, carry):
        lt = jax.lax.broadcast(t * LANES, (LANES,)) + lane
        lt64 = lt * NE
        lt8 = lt * TOPK

        gs = []
        for g in range(NG):
            scl = []
            for j in range(GSZ):
                e = g * GSZ + j
                x = plsc.load_gather(chunk, [lt64 + e])
                sg = 1.0 / (1.0 + jnp.exp(-x))
                scores_t[pl.ds(e * LANES, LANES)] = sg
                scv = sg + bias_vm[pl.ds(e * LANES, LANES)]
                scm_t[pl.ds(e * LANES, LANES)] = scv
                scl.append(scv)
            p1 = [jnp.maximum(scl[2 * k], scl[2 * k + 1]) for k in range(4)]
            p2 = [jnp.minimum(scl[2 * k], scl[2 * k + 1]) for k in range(4)]
            a1, a2 = _top2_combine(p1[0], p2[0], p1[1], p2[1])
            b1, b2 = _top2_combine(p1[2], p2[2], p1[3], p2[3])
            g1, g2 = _top2_combine(a1, a2, b1, b2)
            gs.append(g1 + g2)

        chosen = [None] * NG
        big = jnp.full((LANES,), NG, jnp.int32)
        for _ in range(TOPKG):
            cur = _max_tree(gs)
            gi = _min_tree([
                jnp.where(gs[g] == cur, jnp.full((LANES,), g, jnp.int32), big)
                for g in range(NG)])
            for g in range(NG):
                hit = gi == g
                chosen[g] = hit if chosen[g] is None else (chosen[g] | hit)
                gs[g] = jnp.where(hit, neg_inf, gs[g])

        gmax = []
        gsums = []
        for g in range(NG):
            vals = []
            for j in range(GSZ):
                e = g * GSZ + j
                v = jnp.where(chosen[g], scm_t[pl.ds(e * LANES, LANES)], 0.0)
                scm_t[pl.ds(e * LANES, LANES)] = v
                vals.append(v)
            gmax.append(_max_tree(vals))
            gsums.append(_sum_tree(vals))
        rsum = _sum_tree(gsums)

        rcp = 1.0 / rsum
        for e in range(NE):
            plsc.store_scatter(
                chunk, [lt64 + e],
                scm_t[pl.ds(e * LANES, LANES)] * rcp)

        wvs = []
        bigj = jnp.full((LANES,), GSZ, jnp.int32)
        for r in range(TOPK):
            cur = _max_tree(gmax)
            gsel = _min_tree([
                jnp.where(gmax[g] == cur,
                          jnp.full((LANES,), g, jnp.int32), big)
                for g in range(NG)])
            grow = gsel * (GSZ * LANES) + lane
            cand = [plsc.load_gather(scm_t, [grow + j * LANES])
                    for j in range(GSZ)]
            jsel = _min_tree([
                jnp.where(cand[j] == cur,
                          jnp.full((LANES,), j, jnp.int32), bigj)
                for j in range(GSZ)])
            esel = gsel * GSZ + jsel
            eoff = grow + jsel * LANES
            wvs.append(plsc.load_gather(scores_t, [eoff]))
            plsc.store_scatter(scm_t, [eoff], neg_inf)
            newg = _max_tree([
                jnp.where(jsel == j, neg_inf, cand[j]) for j in range(GSZ)])
            for g in range(NG):
                gmax[g] = jnp.where(gsel == g, newg, gmax[g])
            plsc.store_scatter(oidx, [lt8 + r], esel)
            plsc.addupdate_scatter(counts, [esel], ones)

        factor = SCALE / (_sum_tree(wvs) + 1e-20)
        for r in range(TOPK):
            plsc.store_scatter(ow, [lt8 + r], wvs[r] * factor)
        return carry

    lax.fori_loop(0, NT, tile_body, 0, unroll=False)

    pltpu.sync_copy(chunk, rw_hbm.at[pl.ds(base * NE, TPW * NE)])
    pltpu.sync_copy(oidx, ti_hbm.at[pl.ds(base * TOPK, TPW * TOPK)])
    pltpu.sync_copy(ow, tw_hbm.at[pl.ds(base * TOPK, TPW * TOPK)])
    pltpu.sync_copy(counts, hist_hbm.at[wid])


_router = functools.partial(
    pl.kernel,
    out_type=[
        jax.ShapeDtypeStruct((SEQ * NE,), jnp.float32),
        jax.ShapeDtypeStruct((SEQ * TOPK,), jnp.float32),
        jax.ShapeDtypeStruct((SEQ * TOPK,), jnp.int32),
        jax.ShapeDtypeStruct((NW, NE), jnp.float32),
    ],
    mesh=plsc.VectorSubcoreMesh(core_axis_name="c", subcore_axis_name="s"),
    compiler_params=pltpu.CompilerParams(needs_layout_passes=False),
    scratch_types=[
        pltpu.VMEM((TPW * NE,), jnp.float32),
        pltpu.VMEM((NE * LANES,), jnp.float32),
        pltpu.VMEM((NE * LANES,), jnp.float32),
        pltpu.VMEM((NE * LANES,), jnp.float32),
        pltpu.VMEM((TPW * TOPK,), jnp.int32),
        pltpu.VMEM((TPW * TOPK,), jnp.float32),
        pltpu.VMEM((NE,), jnp.float32),
    ],
)(_router_body)


@jax.jit
def kernel(logits, e_score_correction_bias):
    bias_b = jnp.broadcast_to(
        e_score_correction_bias.astype(jnp.float32)[:, None],
        (NE, LANES)).reshape(-1)
    rw, tw, ti, hist = _router(logits.reshape(-1), bias_b)
    return (logits,
            rw.reshape(SEQ, NE),
            tw.reshape(SEQ, TOPK),
            ti.reshape(SEQ, TOPK),
            hist.sum(axis=0))


# speedup vs baseline: 3.4964x; 1.5369x over previous
import functools

import jax
import jax.numpy as jnp
from jax import lax
from jax.experimental import pallas as pl
from jax.experimental.pallas import tpu as pltpu
from jax.experimental.pallas import tpu_sc as plsc

SEQ = 16384
NE = 64
NG = 8
GSZ = NE // NG
TOPK = 8
TOPKG = 4
SCALE = 2.5
LANES = 16

_INFO = plsc.get_sparse_core_info()
NC = _INFO.num_cores
NS = _INFO.num_subcores
NW = NC * NS
TPW = SEQ // NW
NT = TPW // LANES


def _top2_combine(x1, x2, y1, y2):
    return jnp.maximum(x1, y1), jnp.maximum(jnp.minimum(x1, y1),
                                            jnp.maximum(x2, y2))


def _tree(vs, op):
    vs = list(vs)
    while len(vs) > 1:
        nxt = [op(vs[i], vs[i + 1]) for i in range(0, len(vs) - 1, 2)]
        if len(vs) % 2:
            nxt.append(vs[-1])
        vs = nxt
    return vs[0]


def _max_tree(vs):
    return _tree(vs, jnp.maximum)


def _min_tree(vs):
    return _tree(vs, jnp.minimum)


def _sum_tree(vs):
    return _tree(vs, lambda a, b: a + b)


def _router_body(logits_hbm, bias_hbm, rw_hbm, tw_hbm, ti_hbm, hist_hbm,
                 chunk, scm_t, bias_vm, oidx, ow, counts):
    c = lax.axis_index("c")
    s = lax.axis_index("s")
    wid = s * NC + c
    base = wid * TPW

    pltpu.sync_copy(logits_hbm.at[pl.ds(base * NE, TPW * NE)], chunk)
    pltpu.sync_copy(bias_hbm, bias_vm)

    lane = lax.iota(jnp.int32, LANES)
    zeros = jnp.zeros((LANES,), jnp.float32)
    ones = jnp.ones((LANES,), jnp.float32)
    neg_inf = jnp.full((LANES,), -jnp.inf, jnp.float32)
    big = jnp.full((LANES,), NG, jnp.int32)
    bigj = jnp.full((LANES,), GSZ, jnp.int32)
    for i in range(NE // LANES):
        counts[pl.ds(i * LANES, LANES)] = zeros

    def tile_body(t, carry):
        lt = jax.lax.broadcast(t * LANES, (LANES,)) + lane
        lt64 = lt * NE
        lt8 = lt * TOPK

        gs = []
        gsum = []
        for g in range(NG):
            es = [g * GSZ + j for j in range(GSZ)]
            bs = [bias_vm[pl.ds(e * LANES, LANES)] for e in es]
            xs = [plsc.load_gather(chunk, [lt64 + e]) for e in es]
            ex = [jnp.exp(-x) for x in xs]
            dn = [1.0 + v for v in ex]
            rc = [1.0 / d for d in dn]
            scl = [rc[j] + bs[j] for j in range(GSZ)]
            for j in range(GSZ):
                scm_t[pl.ds(es[j] * LANES, LANES)] = scl[j]
            p1 = [jnp.maximum(scl[2 * k], scl[2 * k + 1]) for k in range(4)]
            p2 = [jnp.minimum(scl[2 * k], scl[2 * k + 1]) for k in range(4)]
            a1, a2 = _top2_combine(p1[0], p2[0], p1[1], p2[1])
            b1, b2 = _top2_combine(p1[2], p2[2], p1[3], p2[3])
            g1, g2 = _top2_combine(a1, a2, b1, b2)
            gs.append(g1 + g2)
            gsum.append(_sum_tree(scl))

        chosen = [None] * NG
        for _ in range(TOPKG):
            cur = _max_tree(gs)
            gi = _min_tree([
                jnp.where(gs[g] == cur, jnp.full((LANES,), g, jnp.int32), big)
                for g in range(NG)])
            for g in range(NG):
                hit = gi == g
                chosen[g] = hit if chosen[g] is None else (chosen[g] | hit)
                gs[g] = jnp.where(hit, neg_inf, gs[g])

        rsum = _sum_tree([jnp.where(chosen[g], gsum[g], 0.0)
                          for g in range(NG)])
        rcp = 1.0 / rsum

        gmax = []
        for g in range(NG):
            raw = [scm_t[pl.ds((g * GSZ + j) * LANES, LANES)]
                   for j in range(GSZ)]
            vals = [jnp.where(chosen[g], v, 0.0) for v in raw]
            for j in range(GSZ):
                scm_t[pl.ds((g * GSZ + j) * LANES, LANES)] = vals[j]
            rws = [v * rcp for v in vals]
            for j in range(GSZ):
                plsc.store_scatter(chunk, [lt64 + g * GSZ + j], rws[j])
            gmax.append(_max_tree(vals))

        wvs = []
        for r in range(TOPK):
            cur = _max_tree(gmax)
            gsel = _min_tree([
                jnp.where(gmax[g] == cur,
                          jnp.full((LANES,), g, jnp.int32), big)
                for g in range(NG)])
            grow = gsel * (GSZ * LANES) + lane
            cand = [plsc.load_gather(scm_t, [grow + j * LANES])
                    for j in range(GSZ)]
            jsel = _min_tree([
                jnp.where(cand[j] == cur,
                          jnp.full((LANES,), j, jnp.int32), bigj)
                for j in range(GSZ)])
            esel = gsel * GSZ + jsel
            eoff = grow + jsel * LANES
            wvs.append(plsc.load_gather(scm_t, [eoff])
                       - plsc.load_gather(bias_vm, [eoff]))
            plsc.store_scatter(scm_t, [eoff], neg_inf)
            newg = _max_tree([
                jnp.where(jsel == j, neg_inf, cand[j]) for j in range(GSZ)])
            for g in range(NG):
                gmax[g] = jnp.where(gsel == g, newg, gmax[g])
            plsc.store_scatter(oidx, [lt8 + r], esel)
            plsc.addupdate_scatter(counts, [esel], ones)

        factor = SCALE / (_sum_tree(wvs) + 1e-20)
        for r in range(TOPK):
            plsc.store_scatter(ow, [lt8 + r], wvs[r] * factor)
        return carry

    lax.fori_loop(0, NT, tile_body, 0, unroll=False)

    pltpu.sync_copy(chunk, rw_hbm.at[pl.ds(base * NE, TPW * NE)])
    pltpu.sync_copy(oidx, ti_hbm.at[pl.ds(base * TOPK, TPW * TOPK)])
    pltpu.sync_copy(ow, tw_hbm.at[pl.ds(base * TOPK, TPW * TOPK)])
    pltpu.sync_copy(counts, hist_hbm.at[wid])


_router = functools.partial(
    pl.kernel,
    out_type=[
        jax.ShapeDtypeStruct((SEQ * NE,), jnp.float32),
        jax.ShapeDtypeStruct((SEQ * TOPK,), jnp.float32),
        jax.ShapeDtypeStruct((SEQ * TOPK,), jnp.int32),
        jax.ShapeDtypeStruct((NW, NE), jnp.float32),
    ],
    mesh=plsc.VectorSubcoreMesh(core_axis_name="c", subcore_axis_name="s"),
    compiler_params=pltpu.CompilerParams(needs_layout_passes=False),
    scratch_types=[
        pltpu.VMEM((TPW * NE,), jnp.float32),
        pltpu.VMEM((NE * LANES,), jnp.float32),
        pltpu.VMEM((NE * LANES,), jnp.float32),
        pltpu.VMEM((TPW * TOPK,), jnp.int32),
        pltpu.VMEM((TPW * TOPK,), jnp.float32),
        pltpu.VMEM((NE,), jnp.float32),
    ],
)(_router_body)


@jax.jit
def kernel(logits, e_score_correction_bias):
    bias_b = jnp.broadcast_to(
        e_score_correction_bias.astype(jnp.float32)[:, None],
        (NE, LANES)).reshape(-1)
    rw, tw, ti, hist = _router(logits.reshape(-1), bias_b)
    return (logits,
            rw.reshape(SEQ, NE),
            tw.reshape(SEQ, TOPK),
            ti.reshape(SEQ, TOPK),
            hist.sum(axis=0))


# speedup vs baseline: 3.5069x; 1.0030x over previous
import functools

import jax
import jax.numpy as jnp
from jax import lax
from jax.experimental import pallas as pl
from jax.experimental.pallas import tpu as pltpu
from jax.experimental.pallas import tpu_sc as plsc

SEQ = 16384
NE = 64
NG = 8
GSZ = NE // NG
TOPK = 8
TOPKG = 4
SCALE = 2.5
LANES = 16

_INFO = plsc.get_sparse_core_info()
NC = _INFO.num_cores
NS = _INFO.num_subcores
NW = NC * NS
TPW = SEQ // NW
NT = TPW // LANES


def _top2_combine(x1, x2, y1, y2):
    return jnp.maximum(x1, y1), jnp.maximum(jnp.minimum(x1, y1),
                                            jnp.maximum(x2, y2))


def _tree(vs, op):
    vs = list(vs)
    while len(vs) > 1:
        nxt = [op(vs[i], vs[i + 1]) for i in range(0, len(vs) - 1, 2)]
        if len(vs) % 2:
            nxt.append(vs[-1])
        vs = nxt
    return vs[0]


def _max_tree(vs):
    return _tree(vs, jnp.maximum)


def _min_tree(vs):
    return _tree(vs, jnp.minimum)


def _sum_tree(vs):
    return _tree(vs, lambda a, b: a + b)


def _router_body(logits_hbm, bias_hbm, rw_hbm, tw_hbm, ti_hbm, hist_hbm,
                 chunk, scm_t, bias_vm, oidx, ow, counts):
    c = lax.axis_index("c")
    s = lax.axis_index("s")
    wid = s * NC + c
    base = wid * TPW

    pltpu.sync_copy(logits_hbm.at[pl.ds(base, TPW), :], chunk)
    pltpu.sync_copy(bias_hbm, bias_vm)

    lane = lax.iota(jnp.int32, LANES)
    zeros = jnp.zeros((LANES,), jnp.float32)
    ones = jnp.ones((LANES,), jnp.float32)
    neg_inf = jnp.full((LANES,), -jnp.inf, jnp.float32)
    big = jnp.full((LANES,), NG, jnp.int32)
    bigj = jnp.full((LANES,), GSZ, jnp.int32)
    for i in range(NE // LANES):
        counts[pl.ds(i * LANES, LANES)] = zeros

    def tile_body(t, carry):
        lt = jax.lax.broadcast(t * LANES, (LANES,)) + lane

        gs = []
        gsum = []
        for g in range(NG):
            es = [g * GSZ + j for j in range(GSZ)]
            bs = [bias_vm[e] for e in es]
            xs = [plsc.load_gather(
                chunk, [lt, jnp.full((LANES,), e, jnp.int32)]) for e in es]
            ex = [jnp.exp(-x) for x in xs]
            dn = [1.0 + v for v in ex]
            rc = [1.0 / d for d in dn]
            scl = [rc[j] + bs[j] for j in range(GSZ)]
            for j in range(GSZ):
                scm_t[pl.ds(es[j] * LANES, LANES)] = scl[j]
            p1 = [jnp.maximum(scl[2 * k], scl[2 * k + 1]) for k in range(4)]
            p2 = [jnp.minimum(scl[2 * k], scl[2 * k + 1]) for k in range(4)]
            a1, a2 = _top2_combine(p1[0], p2[0], p1[1], p2[1])
            b1, b2 = _top2_combine(p1[2], p2[2], p1[3], p2[3])
            g1, g2 = _top2_combine(a1, a2, b1, b2)
            gs.append(g1 + g2)
            gsum.append(_sum_tree(scl))

        chosen = [None] * NG
        for _ in range(TOPKG):
            cur = _max_tree(gs)
            gi = _min_tree([
                jnp.where(gs[g] == cur, jnp.full((LANES,), g, jnp.int32), big)
                for g in range(NG)])
            for g in range(NG):
                hit = gi == g
                chosen[g] = hit if chosen[g] is None else (chosen[g] | hit)
                gs[g] = jnp.where(hit, neg_inf, gs[g])

        rsum = _sum_tree([jnp.where(chosen[g], gsum[g], 0.0)
                          for g in range(NG)])
        rcp = 1.0 / rsum

        gmax = []
        for g in range(NG):
            raw = [scm_t[pl.ds((g * GSZ + j) * LANES, LANES)]
                   for j in range(GSZ)]
            vals = [jnp.where(chosen[g], v, 0.0) for v in raw]
            for j in range(GSZ):
                scm_t[pl.ds((g * GSZ + j) * LANES, LANES)] = vals[j]
            rws = [v * rcp for v in vals]
            for j in range(GSZ):
                plsc.store_scatter(
                    chunk,
                    [lt, jnp.full((LANES,), g * GSZ + j, jnp.int32)],
                    rws[j])
            gmax.append(_max_tree(vals))

        wvs = []
        for r in range(TOPK):
            cur = _max_tree(gmax)
            gsel = _min_tree([
                jnp.where(gmax[g] == cur,
                          jnp.full((LANES,), g, jnp.int32), big)
                for g in range(NG)])
            grow = gsel * (GSZ * LANES) + lane
            cand = [plsc.load_gather(scm_t, [grow + j * LANES])
                    for j in range(GSZ)]
            jsel = _min_tree([
                jnp.where(cand[j] == cur,
                          jnp.full((LANES,), j, jnp.int32), bigj)
                for j in range(GSZ)])
            esel = gsel * GSZ + jsel
            eoff = grow + jsel * LANES
            wvs.append(plsc.load_gather(scm_t, [eoff])
                       - plsc.load_gather(bias_vm, [esel, lane]))
            plsc.store_scatter(scm_t, [eoff], neg_inf)
            newg = _max_tree([
                jnp.where(jsel == j, neg_inf, cand[j]) for j in range(GSZ)])
            for g in range(NG):
                gmax[g] = jnp.where(gsel == g, newg, gmax[g])
            plsc.store_scatter(
                oidx, [lt, jnp.full((LANES,), r, jnp.int32)], esel)
            plsc.addupdate_scatter(counts, [esel], ones)

        factor = SCALE / (_sum_tree(wvs) + 1e-20)
        for r in range(TOPK):
            plsc.store_scatter(
                ow, [lt, jnp.full((LANES,), r, jnp.int32)], wvs[r] * factor)
        return carry

    lax.fori_loop(0, NT, tile_body, 0, unroll=False)

    pltpu.sync_copy(chunk, rw_hbm.at[pl.ds(base, TPW), :])
    pltpu.sync_copy(oidx, ti_hbm.at[pl.ds(base, TPW), :])
    pltpu.sync_copy(ow, tw_hbm.at[pl.ds(base, TPW), :])
    pltpu.sync_copy(counts, hist_hbm.at[wid])


_router = functools.partial(
    pl.kernel,
    out_type=[
        jax.ShapeDtypeStruct((SEQ, NE), jnp.float32),
        jax.ShapeDtypeStruct((SEQ, TOPK), jnp.float32),
        jax.ShapeDtypeStruct((SEQ, TOPK), jnp.int32),
        jax.ShapeDtypeStruct((NW, NE), jnp.float32),
    ],
    mesh=plsc.VectorSubcoreMesh(core_axis_name="c", subcore_axis_name="s"),
    compiler_params=pltpu.CompilerParams(needs_layout_passes=False,
                                         use_tc_tiling_on_sc=False),
    scratch_types=[
        pltpu.VMEM((TPW, NE), jnp.float32),
        pltpu.VMEM((NE * LANES,), jnp.float32),
        pltpu.VMEM((NE, LANES), jnp.float32),
        pltpu.VMEM((TPW, TOPK), jnp.int32),
        pltpu.VMEM((TPW, TOPK), jnp.float32),
        pltpu.VMEM((NE,), jnp.float32),
    ],
)(_router_body)


@jax.jit
def kernel(logits, e_score_correction_bias):
    bias_b = jnp.broadcast_to(
        e_score_correction_bias.astype(jnp.float32)[:, None], (NE, LANES))
    rw, tw, ti, hist = _router(logits, bias_b)
    return (logits, rw, tw, ti, hist.sum(axis=0))


# speedup vs baseline: 3.5261x; 1.0055x over previous
import functools

import jax
import jax.numpy as jnp
from jax import lax
from jax.experimental import pallas as pl
from jax.experimental.pallas import tpu as pltpu
from jax.experimental.pallas import tpu_sc as plsc

SEQ = 16384
NE = 64
NG = 8
GSZ = NE // NG
TOPK = 8
TOPKG = 4
SCALE = 2.5
LANES = 16

_INFO = plsc.get_sparse_core_info()
NC = _INFO.num_cores
NS = _INFO.num_subcores
NW = NC * NS
TPW = SEQ // NW
NT = TPW // LANES


def _top2_combine(x1, x2, y1, y2):
    return jnp.maximum(x1, y1), jnp.maximum(jnp.minimum(x1, y1),
                                            jnp.maximum(x2, y2))


def _tree(vs, op):
    vs = list(vs)
    while len(vs) > 1:
        nxt = [op(vs[i], vs[i + 1]) for i in range(0, len(vs) - 1, 2)]
        if len(vs) % 2:
            nxt.append(vs[-1])
        vs = nxt
    return vs[0]


def _max_tree(vs):
    return _tree(vs, jnp.maximum)


def _min_tree(vs):
    return _tree(vs, jnp.minimum)


def _sum_tree(vs):
    return _tree(vs, lambda a, b: a + b)


def _router_body(logits_hbm, bias_hbm, rw_hbm, tw_hbm, ti_hbm, hist_hbm,
                 chunk, scm_t, bias_vm, oidx, ow, counts):
    c = lax.axis_index("c")
    s = lax.axis_index("s")
    wid = s * NC + c
    base = wid * TPW

    pltpu.sync_copy(logits_hbm.at[pl.ds(base, TPW), :], chunk)
    pltpu.sync_copy(bias_hbm, bias_vm)

    lane = lax.iota(jnp.int32, LANES)
    zeros = jnp.zeros((LANES,), jnp.float32)
    ones = jnp.ones((LANES,), jnp.float32)
    neg_inf = jnp.full((LANES,), -jnp.inf, jnp.float32)
    big = jnp.full((LANES,), NG, jnp.int32)
    bigj = jnp.full((LANES,), GSZ, jnp.int32)
    for i in range(NE // LANES):
        counts[pl.ds(i * LANES, LANES)] = zeros

    def tile_body(t, carry):
        lt = jax.lax.broadcast(t * LANES, (LANES,)) + lane

        gs = []
        gtop = []
        gsum = []
        for g in range(NG):
            es = [g * GSZ + j for j in range(GSZ)]
            bs = [bias_vm[e] for e in es]
            xs = [plsc.load_gather(
                chunk, [lt, jnp.full((LANES,), e, jnp.int32)]) for e in es]
            ex = [jnp.exp(-x) for x in xs]
            dn = [1.0 + v for v in ex]
            rc = [1.0 / d for d in dn]
            scl = [rc[j] + bs[j] for j in range(GSZ)]
            for j in range(GSZ):
                scm_t[pl.ds(es[j] * LANES, LANES)] = scl[j]
            p1 = [jnp.maximum(scl[2 * k], scl[2 * k + 1]) for k in range(4)]
            p2 = [jnp.minimum(scl[2 * k], scl[2 * k + 1]) for k in range(4)]
            a1, a2 = _top2_combine(p1[0], p2[0], p1[1], p2[1])
            b1, b2 = _top2_combine(p1[2], p2[2], p1[3], p2[3])
            g1, g2 = _top2_combine(a1, a2, b1, b2)
            gs.append(g1 + g2)
            gtop.append(g1)
            gsum.append(_sum_tree(scl))

        chosen = [None] * NG
        for _ in range(TOPKG):
            cur = _max_tree(gs)
            gi = _min_tree([
                jnp.where(gs[g] == cur, jnp.full((LANES,), g, jnp.int32), big)
                for g in range(NG)])
            for g in range(NG):
                hit = gi == g
                chosen[g] = hit if chosen[g] is None else (chosen[g] | hit)
                gs[g] = jnp.where(hit, neg_inf, gs[g])

        rsum = _sum_tree([jnp.where(chosen[g], gsum[g], 0.0)
                          for g in range(NG)])
        rcp = 1.0 / rsum

        gmax = [jnp.where(chosen[g], gtop[g], 0.0) for g in range(NG)]
        for g in range(NG):
            raw = [scm_t[pl.ds((g * GSZ + j) * LANES, LANES)]
                   for j in range(GSZ)]
            rws = [jnp.where(chosen[g], v, 0.0) * rcp for v in raw]
            for j in range(GSZ):
                plsc.store_scatter(
                    chunk,
                    [lt, jnp.full((LANES,), g * GSZ + j, jnp.int32)],
                    rws[j])

        wvs = []
        for r in range(TOPK):
            cur = _max_tree(gmax)
            gsel = _min_tree([
                jnp.where(gmax[g] == cur,
                          jnp.full((LANES,), g, jnp.int32), big)
                for g in range(NG)])
            grow = gsel * (GSZ * LANES) + lane
            cand = [plsc.load_gather(scm_t, [grow + j * LANES])
                    for j in range(GSZ)]
            jsel = _min_tree([
                jnp.where(cand[j] == cur,
                          jnp.full((LANES,), j, jnp.int32), bigj)
                for j in range(GSZ)])
            esel = gsel * GSZ + jsel
            eoff = grow + jsel * LANES
            wvs.append(plsc.load_gather(scm_t, [eoff])
                       - plsc.load_gather(bias_vm, [esel, lane]))
            plsc.store_scatter(scm_t, [eoff], neg_inf)
            newg = _max_tree([
                jnp.where(jsel == j, neg_inf, cand[j]) for j in range(GSZ)])
            for g in range(NG):
                gmax[g] = jnp.where(gsel == g, newg, gmax[g])
            plsc.store_scatter(
                oidx, [lt, jnp.full((LANES,), r, jnp.int32)], esel)
            plsc.addupdate_scatter(counts, [esel], ones)

        factor = SCALE / (_sum_tree(wvs) + 1e-20)
        for r in range(TOPK):
            plsc.store_scatter(
                ow, [lt, jnp.full((LANES,), r, jnp.int32)], wvs[r] * factor)
        return carry

    lax.fori_loop(0, NT, tile_body, 0, unroll=False)

    pltpu.sync_copy(chunk, rw_hbm.at[pl.ds(base, TPW), :])
    pltpu.sync_copy(oidx, ti_hbm.at[pl.ds(base, TPW), :])
    pltpu.sync_copy(ow, tw_hbm.at[pl.ds(base, TPW), :])
    pltpu.sync_copy(counts, hist_hbm.at[wid])


_router = functools.partial(
    pl.kernel,
    out_type=[
        jax.ShapeDtypeStruct((SEQ, NE), jnp.float32),
        jax.ShapeDtypeStruct((SEQ, TOPK), jnp.float32),
        jax.ShapeDtypeStruct((SEQ, TOPK), jnp.int32),
        jax.ShapeDtypeStruct((NW, NE), jnp.float32),
    ],
    mesh=plsc.VectorSubcoreMesh(core_axis_name="c", subcore_axis_name="s"),
    compiler_params=pltpu.CompilerParams(needs_layout_passes=False,
                                         use_tc_tiling_on_sc=False),
    scratch_types=[
        pltpu.VMEM((TPW, NE), jnp.float32),
        pltpu.VMEM((NE * LANES,), jnp.float32),
        pltpu.VMEM((NE, LANES), jnp.float32),
        pltpu.VMEM((TPW, TOPK), jnp.int32),
        pltpu.VMEM((TPW, TOPK), jnp.float32),
        pltpu.VMEM((NE,), jnp.float32),
    ],
)(_router_body)


@jax.jit
def kernel(logits, e_score_correction_bias):
    bias_b = jnp.broadcast_to(
        e_score_correction_bias.astype(jnp.float32)[:, None], (NE, LANES))
    rw, tw, ti, hist = _router(logits, bias_b)
    return (logits, rw, tw, ti, hist.sum(axis=0))


# speedup vs baseline: 6.1868x; 1.7546x over previous
import functools

import jax
import jax.numpy as jnp
from jax import lax
from jax.experimental import pallas as pl
from jax.experimental.pallas import tpu as pltpu
from jax.experimental.pallas import tpu_sc as plsc

SEQ = 16384
NE = 64
NG = 8
GSZ = NE // NG
TOPK = 8
TOPKG = 4
SCALE = 2.5
LANES = 16

_INFO = plsc.get_sparse_core_info()
NC = _INFO.num_cores
NS = _INFO.num_subcores
NW = NC * NS
TPW = SEQ // NW
NT = TPW // LANES


def _top2_combine(x1, x2, y1, y2):
    return jnp.maximum(x1, y1), jnp.maximum(jnp.minimum(x1, y1),
                                            jnp.maximum(x2, y2))


def _tree(vs, op):
    vs = list(vs)
    while len(vs) > 1:
        nxt = [op(vs[i], vs[i + 1]) for i in range(0, len(vs) - 1, 2)]
        if len(vs) % 2:
            nxt.append(vs[-1])
        vs = nxt
    return vs[0]


def _max_tree(vs):
    return _tree(vs, jnp.maximum)


def _min_tree(vs):
    return _tree(vs, jnp.minimum)


def _sum_tree(vs):
    return _tree(vs, lambda a, b: a + b)


def _router_body(lgt_hbm, bias_hbm, rwt_hbm, twt_hbm, tit_hbm, hist_hbm,
                 chunk, scm_t, bias_vm, oidx, ow, counts):
    c = lax.axis_index("c")
    s = lax.axis_index("s")
    wid = s * NC + c

    pltpu.sync_copy(lgt_hbm.at[wid], chunk)
    pltpu.sync_copy(bias_hbm, bias_vm)

    lane = lax.iota(jnp.int32, LANES)
    zeros = jnp.zeros((LANES,), jnp.float32)
    ones = jnp.ones((LANES,), jnp.float32)
    neg_inf = jnp.full((LANES,), -jnp.inf, jnp.float32)
    big = jnp.full((LANES,), NG, jnp.int32)
    bigj = jnp.full((LANES,), GSZ, jnp.int32)
    for i in range(NE // LANES):
        counts[pl.ds(i * LANES, LANES)] = zeros

    def tile_body(t, carry):
        t16 = t * LANES

        gs = []
        gtop = []
        gsum = []
        for g in range(NG):
            es = [g * GSZ + j for j in range(GSZ)]
            bs = [bias_vm[pl.ds(e * LANES, LANES)] for e in es]
            xs = [chunk[e, pl.ds(t16, LANES)] for e in es]
            ex = [jnp.exp(-x) for x in xs]
            dn = [1.0 + v for v in ex]
            rc = [1.0 / d for d in dn]
            scl = [rc[j] + bs[j] for j in range(GSZ)]
            for j in range(GSZ):
                scm_t[pl.ds(es[j] * LANES, LANES)] = scl[j]
            p1 = [jnp.maximum(scl[2 * k], scl[2 * k + 1]) for k in range(4)]
            p2 = [jnp.minimum(scl[2 * k], scl[2 * k + 1]) for k in range(4)]
            a1, a2 = _top2_combine(p1[0], p2[0], p1[1], p2[1])
            b1, b2 = _top2_combine(p1[2], p2[2], p1[3], p2[3])
            g1, g2 = _top2_combine(a1, a2, b1, b2)
            gs.append(g1 + g2)
            gtop.append(g1)
            gsum.append(_sum_tree(scl))

        chosen = [None] * NG
        for _ in range(TOPKG):
            cur = _max_tree(gs)
            gi = _min_tree([
                jnp.where(gs[g] == cur, jnp.full((LANES,), g, jnp.int32), big)
                for g in range(NG)])
            for g in range(NG):
                hit = gi == g
                chosen[g] = hit if chosen[g] is None else (chosen[g] | hit)
                gs[g] = jnp.where(hit, neg_inf, gs[g])

        rsum = _sum_tree([jnp.where(chosen[g], gsum[g], 0.0)
                          for g in range(NG)])
        rcp = 1.0 / rsum

        gmax = [jnp.where(chosen[g], gtop[g], 0.0) for g in range(NG)]
        for g in range(NG):
            for j in range(GSZ):
                e = g * GSZ + j
                v = scm_t[pl.ds(e * LANES, LANES)]
                chunk[e, pl.ds(t16, LANES)] = (
                    jnp.where(chosen[g], v, 0.0) * rcp)

        wvs = []
        for r in range(TOPK):
            cur = _max_tree(gmax)
            gsel = _min_tree([
                jnp.where(gmax[g] == cur,
                          jnp.full((LANES,), g, jnp.int32), big)
                for g in range(NG)])
            grow = gsel * (GSZ * LANES) + lane
            cand = [plsc.load_gather(scm_t, [grow + j * LANES])
                    for j in range(GSZ)]
            jsel = _min_tree([
                jnp.where(cand[j] == cur,
                          jnp.full((LANES,), j, jnp.int32), bigj)
                for j in range(GSZ)])
            esel = gsel * GSZ + jsel
            eoff = grow + jsel * LANES
            wvs.append(plsc.load_gather(scm_t, [eoff])
                       - plsc.load_gather(bias_vm, [eoff]))
            plsc.store_scatter(scm_t, [eoff], neg_inf)
            newg = _max_tree([
                jnp.where(jsel == j, neg_inf, cand[j]) for j in range(GSZ)])
            for g in range(NG):
                gmax[g] = jnp.where(gsel == g, newg, gmax[g])
            oidx[r, pl.ds(t16, LANES)] = esel
            plsc.addupdate_scatter(counts, [esel], ones)

        factor = SCALE / (_sum_tree(wvs) + 1e-20)
        for r in range(TOPK):
            ow[r, pl.ds(t16, LANES)] = wvs[r] * factor
        return carry

    lax.fori_loop(0, NT, tile_body, 0, unroll=False)

    pltpu.sync_copy(chunk, rwt_hbm.at[wid])
    pltpu.sync_copy(oidx, tit_hbm.at[wid])
    pltpu.sync_copy(ow, twt_hbm.at[wid])
    pltpu.sync_copy(counts, hist_hbm.at[wid])


_router = functools.partial(
    pl.kernel,
    out_type=[
        jax.ShapeDtypeStruct((NW, NE, TPW), jnp.float32),
        jax.ShapeDtypeStruct((NW, TOPK, TPW), jnp.float32),
        jax.ShapeDtypeStruct((NW, TOPK, TPW), jnp.int32),
        jax.ShapeDtypeStruct((NW, NE), jnp.float32),
    ],
    mesh=plsc.VectorSubcoreMesh(core_axis_name="c", subcore_axis_name="s"),
    compiler_params=pltpu.CompilerParams(needs_layout_passes=False,
                                         use_tc_tiling_on_sc=False),
    scratch_types=[
        pltpu.VMEM((NE, TPW), jnp.float32),
        pltpu.VMEM((NE * LANES,), jnp.float32),
        pltpu.VMEM((NE * LANES,), jnp.float32),
        pltpu.VMEM((TOPK, TPW), jnp.int32),
        pltpu.VMEM((TOPK, TPW), jnp.float32),
        pltpu.VMEM((NE,), jnp.float32),
    ],
)(_router_body)


@jax.jit
def kernel(logits, e_score_correction_bias):
    bias_b = jnp.broadcast_to(
        e_score_correction_bias.astype(jnp.float32)[:, None],
        (NE, LANES)).reshape(-1)
    lgt = jnp.transpose(logits.reshape(NW, TPW, NE), (0, 2, 1))
    rwt, twt, tit, hist = _router(lgt, bias_b)
    rw = jnp.transpose(rwt, (0, 2, 1)).reshape(SEQ, NE)
    tw = jnp.transpose(twt, (0, 2, 1)).reshape(SEQ, TOPK)
    ti = jnp.transpose(tit, (0, 2, 1)).reshape(SEQ, TOPK)
    return (logits, rw, tw, ti, hist.sum(axis=0))


# speedup vs baseline: 7.0032x; 1.1320x over previous
import functools

import jax
import jax.numpy as jnp
from jax import lax
from jax.experimental import pallas as pl
from jax.experimental.pallas import tpu as pltpu
from jax.experimental.pallas import tpu_sc as plsc

SEQ = 16384
NE = 64
NG = 8
GSZ = NE // NG
TOPK = 8
TOPKG = 4
SCALE = 2.5
LANES = 16

_INFO = plsc.get_sparse_core_info()
NC = _INFO.num_cores
NS = _INFO.num_subcores
NW = NC * NS
TPW = SEQ // NW
NT = TPW // LANES


def _top2_combine(x1, x2, y1, y2):
    return jnp.maximum(x1, y1), jnp.maximum(jnp.minimum(x1, y1),
                                            jnp.maximum(x2, y2))


def _tree(vs, op):
    vs = list(vs)
    while len(vs) > 1:
        nxt = [op(vs[i], vs[i + 1]) for i in range(0, len(vs) - 1, 2)]
        if len(vs) % 2:
            nxt.append(vs[-1])
        vs = nxt
    return vs[0]


def _max_tree(vs):
    return _tree(vs, jnp.maximum)


def _min_tree(vs):
    return _tree(vs, jnp.minimum)


def _sum_tree(vs):
    return _tree(vs, lambda a, b: a + b)


def _router_body(lgt_hbm, bias_hbm, rwt_hbm, twt_hbm, tit_hbm, hist_hbm,
                 chunk, scm_t, bias_vm, oidx, ow, counts):
    c = lax.axis_index("c")
    s = lax.axis_index("s")
    wid = s * NC + c

    pltpu.sync_copy(lgt_hbm.at[wid], chunk)
    pltpu.sync_copy(bias_hbm, bias_vm)

    lane = lax.iota(jnp.int32, LANES)
    zeros = jnp.zeros((LANES,), jnp.float32)
    ones = jnp.ones((LANES,), jnp.float32)
    neg_inf = jnp.full((LANES,), -jnp.inf, jnp.float32)
    big = jnp.full((LANES,), NG, jnp.int32)
    bigj = jnp.full((LANES,), GSZ, jnp.int32)
    for i in range(NE // LANES):
        counts[pl.ds(i * LANES, LANES)] = zeros

    def tile_body(t, carry):
        t16 = t * LANES

        gs = []
        gtop = []
        gsum = []
        for g in range(NG):
            es = [g * GSZ + j for j in range(GSZ)]
            bs = [bias_vm[pl.ds(e * LANES, LANES)] for e in es]
            xs = [chunk[e, pl.ds(t16, LANES)] for e in es]
            ex = [jnp.exp(-x) for x in xs]
            dn = [1.0 + v for v in ex]
            rc = [1.0 / d for d in dn]
            scl = [rc[j] + bs[j] for j in range(GSZ)]
            for j in range(GSZ):
                scm_t[pl.ds(es[j] * LANES, LANES)] = scl[j]
            p1 = [jnp.maximum(scl[2 * k], scl[2 * k + 1]) for k in range(4)]
            p2 = [jnp.minimum(scl[2 * k], scl[2 * k + 1]) for k in range(4)]
            a1, a2 = _top2_combine(p1[0], p2[0], p1[1], p2[1])
            b1, b2 = _top2_combine(p1[2], p2[2], p1[3], p2[3])
            g1, g2 = _top2_combine(a1, a2, b1, b2)
            gs.append(g1 + g2)
            gtop.append(g1)
            gsum.append(_sum_tree(scl))

        chosen = [None] * NG
        for _ in range(TOPKG):
            cur = _max_tree(gs)
            gi = _min_tree([
                jnp.where(gs[g] == cur, jnp.full((LANES,), g, jnp.int32), big)
                for g in range(NG)])
            for g in range(NG):
                hit = gi == g
                chosen[g] = hit if chosen[g] is None else (chosen[g] | hit)
                gs[g] = jnp.where(hit, neg_inf, gs[g])

        rsum = _sum_tree([jnp.where(chosen[g], gsum[g], 0.0)
                          for g in range(NG)])
        rcp = 1.0 / rsum

        gmax = [jnp.where(chosen[g], gtop[g], 0.0) for g in range(NG)]
        for g in range(NG):
            rg = jnp.where(chosen[g], rcp, 0.0)
            raw = [scm_t[pl.ds((g * GSZ + j) * LANES, LANES)]
                   for j in range(GSZ)]
            rws = [v * rg for v in raw]
            for j in range(GSZ):
                chunk[g * GSZ + j, pl.ds(t16, LANES)] = rws[j]

        wvs = []
        for r in range(TOPK):
            cur = _max_tree(gmax)
            gsel = _min_tree([
                jnp.where(gmax[g] == cur,
                          jnp.full((LANES,), g, jnp.int32), big)
                for g in range(NG)])
            grow = gsel * (GSZ * LANES) + lane
            cand = [plsc.load_gather(scm_t, [grow + j * LANES])
                    for j in range(GSZ)]
            jsel = _min_tree([
                jnp.where(cand[j] == cur,
                          jnp.full((LANES,), j, jnp.int32), bigj)
                for j in range(GSZ)])
            esel = gsel * GSZ + jsel
            eoff = grow + jsel * LANES
            wvs.append(plsc.load_gather(scm_t, [eoff])
                       - plsc.load_gather(bias_vm, [eoff]))
            plsc.store_scatter(scm_t, [eoff], neg_inf)
            newg = _max_tree([
                jnp.where(jsel == j, neg_inf, cand[j]) for j in range(GSZ)])
            for g in range(NG):
                gmax[g] = jnp.where(gsel == g, newg, gmax[g])
            oidx[r, pl.ds(t16, LANES)] = esel
            plsc.addupdate_scatter(counts, [esel], ones)

        factor = SCALE / (_sum_tree(wvs) + 1e-20)
        for r in range(TOPK):
            ow[r, pl.ds(t16, LANES)] = wvs[r] * factor
        return carry

    lax.fori_loop(0, NT, tile_body, 0, unroll=False)

    pltpu.sync_copy(chunk, rwt_hbm.at[wid])
    pltpu.sync_copy(oidx, tit_hbm.at[wid])
    pltpu.sync_copy(ow, twt_hbm.at[wid])
    pltpu.sync_copy(counts, hist_hbm.at[wid])


_router = functools.partial(
    pl.kernel,
    out_type=[
        jax.ShapeDtypeStruct((NW, NE, TPW), jnp.float32),
        jax.ShapeDtypeStruct((NW, TOPK, TPW), jnp.float32),
        jax.ShapeDtypeStruct((NW, TOPK, TPW), jnp.int32),
        jax.ShapeDtypeStruct((NW, NE), jnp.float32),
    ],
    mesh=plsc.VectorSubcoreMesh(core_axis_name="c", subcore_axis_name="s"),
    compiler_params=pltpu.CompilerParams(needs_layout_passes=False,
                                         use_tc_tiling_on_sc=False),
    scratch_types=[
        pltpu.VMEM((NE, TPW), jnp.float32),
        pltpu.VMEM((NE * LANES,), jnp.float32),
        pltpu.VMEM((NE * LANES,), jnp.float32),
        pltpu.VMEM((TOPK, TPW), jnp.int32),
        pltpu.VMEM((TOPK, TPW), jnp.float32),
        pltpu.VMEM((NE,), jnp.float32),
    ],
)(_router_body)


@jax.jit
def kernel(logits, e_score_correction_bias):
    bias_b = jnp.broadcast_to(
        e_score_correction_bias.astype(jnp.float32)[:, None],
        (NE, LANES)).reshape(-1)
    lgt = jnp.transpose(logits.reshape(NW, TPW, NE), (0, 2, 1))
    rwt, twt, tit, hist = _router(lgt, bias_b)
    rw = jnp.transpose(rwt, (0, 2, 1)).reshape(SEQ, NE)
    tw = jnp.transpose(twt, (0, 2, 1)).reshape(SEQ, TOPK)
    ti = jnp.transpose(tit, (0, 2, 1)).reshape(SEQ, TOPK)
    return (logits, rw, tw, ti, hist.sum(axis=0))


# speedup vs baseline: 7.0328x; 1.0042x over previous
import functools

import jax
import jax.numpy as jnp
from jax import lax
from jax.experimental import pallas as pl
from jax.experimental.pallas import tpu as pltpu
from jax.experimental.pallas import tpu_sc as plsc

SEQ = 16384
NE = 64
NG = 8
GSZ = NE // NG
TOPK = 8
TOPKG = 4
SCALE = 2.5
LANES = 16

_INFO = plsc.get_sparse_core_info()
NC = _INFO.num_cores
NS = _INFO.num_subcores
NW = NC * NS
TPW = SEQ // NW
NT = TPW // LANES


def _top2_combine(x1, x2, y1, y2):
    return jnp.maximum(x1, y1), jnp.maximum(jnp.minimum(x1, y1),
                                            jnp.maximum(x2, y2))


def _tree(vs, op):
    vs = list(vs)
    while len(vs) > 1:
        nxt = [op(vs[i], vs[i + 1]) for i in range(0, len(vs) - 1, 2)]
        if len(vs) % 2:
            nxt.append(vs[-1])
        vs = nxt
    return vs[0]


def _max_tree(vs):
    return _tree(vs, jnp.maximum)


def _min_tree(vs):
    return _tree(vs, jnp.minimum)


def _sum_tree(vs):
    return _tree(vs, lambda a, b: a + b)


def _argmax_tree(vs):
    pairs = [(v, jnp.full((LANES,), i, jnp.int32)) for i, v in enumerate(vs)]
    while len(pairs) > 1:
        nxt = []
        for i in range(0, len(pairs) - 1, 2):
            (a, ai), (b, bi) = pairs[i], pairs[i + 1]
            ge = a >= b
            nxt.append((jnp.where(ge, a, b), jnp.where(ge, ai, bi)))
        if len(pairs) % 2:
            nxt.append(pairs[-1])
        pairs = nxt
    return pairs[0]


def _router_body(lgt_hbm, bias_hbm, rwt_hbm, twt_hbm, tit_hbm, hist_hbm,
                 chunk, scm_t, bias_vm, oidx, ow, counts):
    c = lax.axis_index("c")
    s = lax.axis_index("s")
    wid = s * NC + c

    pltpu.sync_copy(lgt_hbm.at[wid], chunk)
    pltpu.sync_copy(bias_hbm, bias_vm)

    lane = lax.iota(jnp.int32, LANES)
    zeros = jnp.zeros((LANES,), jnp.float32)
    ones = jnp.ones((LANES,), jnp.float32)
    neg_inf = jnp.full((LANES,), -jnp.inf, jnp.float32)
    big = jnp.full((LANES,), NG, jnp.int32)
    bigj = jnp.full((LANES,), GSZ, jnp.int32)
    for i in range(NE // LANES):
        counts[pl.ds(i * LANES, LANES)] = zeros

    def tile_body(t, carry):
        t16 = t * LANES

        gs = []
        gtop = []
        gsum = []
        for g in range(NG):
            es = [g * GSZ + j for j in range(GSZ)]
            bs = [bias_vm[pl.ds(e * LANES, LANES)] for e in es]
            xs = [chunk[e, pl.ds(t16, LANES)] for e in es]
            ex = [jnp.exp(-x) for x in xs]
            dn = [1.0 + v for v in ex]
            rc = [1.0 / d for d in dn]
            scl = [rc[j] + bs[j] for j in range(GSZ)]
            for j in range(GSZ):
                scm_t[pl.ds(es[j] * LANES, LANES)] = scl[j]
            p1 = [jnp.maximum(scl[2 * k], scl[2 * k + 1]) for k in range(4)]
            p2 = [jnp.minimum(scl[2 * k], scl[2 * k + 1]) for k in range(4)]
            a1, a2 = _top2_combine(p1[0], p2[0], p1[1], p2[1])
            b1, b2 = _top2_combine(p1[2], p2[2], p1[3], p2[3])
            g1, g2 = _top2_combine(a1, a2, b1, b2)
            gs.append(g1 + g2)
            gtop.append(g1)
            gsum.append(_sum_tree(scl))

        chosen = [None] * NG
        for _ in range(TOPKG):
            _, gi = _argmax_tree(gs)
            for g in range(NG):
                hit = gi == g
                chosen[g] = hit if chosen[g] is None else (chosen[g] | hit)
                gs[g] = jnp.where(hit, neg_inf, gs[g])

        rsum = _sum_tree([jnp.where(chosen[g], gsum[g], 0.0)
                          for g in range(NG)])
        rcp = 1.0 / rsum

        gmax = [jnp.where(chosen[g], gtop[g], 0.0) for g in range(NG)]
        for g in range(NG):
            rg = jnp.where(chosen[g], rcp, 0.0)
            raw = [scm_t[pl.ds((g * GSZ + j) * LANES, LANES)]
                   for j in range(GSZ)]
            rws = [v * rg for v in raw]
            for j in range(GSZ):
                chunk[g * GSZ + j, pl.ds(t16, LANES)] = rws[j]

        wvs = []
        for r in range(TOPK):
            cur, gsel = _argmax_tree(gmax)
            grow = gsel * (GSZ * LANES) + lane
            cand = [plsc.load_gather(scm_t, [grow + j * LANES])
                    for j in range(GSZ)]
            jsel = _min_tree([
                jnp.where(cand[j] == cur,
                          jnp.full((LANES,), j, jnp.int32), bigj)
                for j in range(GSZ)])
            esel = gsel * GSZ + jsel
            eoff = grow + jsel * LANES
            wvs.append(plsc.load_gather(scm_t, [eoff])
                       - plsc.load_gather(bias_vm, [eoff]))
            plsc.store_scatter(scm_t, [eoff], neg_inf)
            newg = _max_tree([
                jnp.where(jsel == j, neg_inf, cand[j]) for j in range(GSZ)])
            for g in range(NG):
                gmax[g] = jnp.where(gsel == g, newg, gmax[g])
            oidx[r, pl.ds(t16, LANES)] = esel
            plsc.addupdate_scatter(counts, [esel], ones)

        factor = SCALE / (_sum_tree(wvs) + 1e-20)
        for r in range(TOPK):
            ow[r, pl.ds(t16, LANES)] = wvs[r] * factor
        return carry

    lax.fori_loop(0, NT, tile_body, 0, unroll=False)

    pltpu.sync_copy(chunk, rwt_hbm.at[wid])
    pltpu.sync_copy(oidx, tit_hbm.at[wid])
    pltpu.sync_copy(ow, twt_hbm.at[wid])
    pltpu.sync_copy(counts, hist_hbm.at[wid])


_router = functools.partial(
    pl.kernel,
    out_type=[
        jax.ShapeDtypeStruct((NW, NE, TPW), jnp.float32),
        jax.ShapeDtypeStruct((NW, TOPK, TPW), jnp.float32),
        jax.ShapeDtypeStruct((NW, TOPK, TPW), jnp.int32),
        jax.ShapeDtypeStruct((NW, NE), jnp.float32),
    ],
    mesh=plsc.VectorSubcoreMesh(core_axis_name="c", subcore_axis_name="s"),
    compiler_params=pltpu.CompilerParams(needs_layout_passes=False,
                                         use_tc_tiling_on_sc=False),
    scratch_types=[
        pltpu.VMEM((NE, TPW), jnp.float32),
        pltpu.VMEM((NE * LANES,), jnp.float32),
        pltpu.VMEM((NE * LANES,), jnp.float32),
        pltpu.VMEM((TOPK, TPW), jnp.int32),
        pltpu.VMEM((TOPK, TPW), jnp.float32),
        pltpu.VMEM((NE,), jnp.float32),
    ],
)(_router_body)


@jax.jit
def kernel(logits, e_score_correction_bias):
    bias_b = jnp.broadcast_to(
        e_score_correction_bias.astype(jnp.float32)[:, None],
        (NE, LANES)).reshape(-1)
    lgt = jnp.transpose(logits.reshape(NW, TPW, NE), (0, 2, 1))
    rwt, twt, tit, hist = _router(lgt, bias_b)
    rw = jnp.transpose(rwt, (0, 2, 1)).reshape(SEQ, NE)
    tw = jnp.transpose(twt, (0, 2, 1)).reshape(SEQ, TOPK)
    ti = jnp.transpose(tit, (0, 2, 1)).reshape(SEQ, TOPK)
    return (logits, rw, tw, ti, hist.sum(axis=0))
